# baseline (speedup 1.0000x reference)
import functools

import jax
import jax.numpy as jnp
from jax import lax
from jax.experimental import pallas as pl
from jax.experimental.pallas import tpu as pltpu
from jax.experimental.pallas import tpu_sc as plsc

_VOCAB = 100000
_EMBED = 64
_BATCH = 1024
_CTX = 20

_NUM_IDX = _BATCH * _CTX
_NC, _NS = 2, 16
_NW = _NC * _NS
_B_PER_W = _NUM_IDX // _NW

_V_BLK = 2048
_GRID_V = (_VOCAB + _V_BLK - 1) // _V_BLK


def _sc_gather(table128, pair_idx):
  mesh = plsc.VectorSubcoreMesh(core_axis_name="c", subcore_axis_name="s")

  @functools.partial(
      pl.kernel,
      mesh=mesh,
      out_type=jax.ShapeDtypeStruct((_NUM_IDX, 2 * _EMBED), jnp.float32),
      scratch_types=[
          pltpu.VMEM((_B_PER_W,), jnp.int32),
          pltpu.VMEM((_B_PER_W, 2 * _EMBED), jnp.float32),
          pltpu.SemaphoreType.DMA,
      ],
  )
  def gather_kernel(table_hbm, idx_hbm, out_hbm, idx_v, rows_v, sem):
    wid = lax.axis_index("s") * _NC + lax.axis_index("c")
    base = wid * _B_PER_W
    pltpu.sync_copy(idx_hbm.at[pl.ds(base, _B_PER_W)], idx_v)
    pltpu.async_copy(table_hbm.at[idx_v], rows_v, sem).wait()
    pltpu.sync_copy(rows_v, out_hbm.at[pl.ds(base, _B_PER_W)])

  return gather_kernel(table128, pair_idx)


def _reduce_body(g_ref, p_ref, h_ref):
  def pick(j):
    g = g_ref[j * _BATCH:(j + 1) * _BATCH, :]
    par = p_ref[j * _BATCH:(j + 1) * _BATCH, :]
    return jnp.where(par > 0.5, g[:, _EMBED:], g[:, :_EMBED])

  acc = pick(0)
  for j in range(1, _CTX):
    acc = acc + pick(j)
  h_ref[...] = acc * (1.0 / _CTX)


def _tc_reduce(gathered, parity):
  return pl.pallas_call(
      _reduce_body,
      out_shape=jax.ShapeDtypeStruct((_BATCH, _EMBED), jnp.float32),
  )(gathered, parity)


def _proj_body(h_ref, w_ref, b_ref, o_ref):
  o_ref[...] = lax.dot_general(
      h_ref[...], w_ref[...],
      dimension_numbers=(((1,), (1,)), ((), ())),
      preferred_element_type=jnp.float32,
  ) + b_ref[...]


def _tc_project(hidden, W, b2):
  return pl.pallas_call(
      _proj_body,
      grid=(_GRID_V,),
      in_specs=[
          pl.BlockSpec((_BATCH, _EMBED), lambda i: (0, 0)),
          pl.BlockSpec((_V_BLK, _EMBED), lambda i: (i, 0)),
          pl.BlockSpec((1, _V_BLK), lambda i: (0, i)),
      ],
      out_specs=pl.BlockSpec((_BATCH, _V_BLK), lambda i: (0, i)),
      out_shape=jax.ShapeDtypeStruct((_BATCH, _VOCAB), jnp.float32),
      compiler_params=pltpu.CompilerParams(
          dimension_semantics=("parallel",),
      ),
  )(hidden, W, b2)


@jax.jit
def kernel(context, emb_table, W, b):
  flat_idx = context.T.reshape(-1)
  pair_idx = lax.shift_right_logical(flat_idx, 1)
  parity = (flat_idx & 1).astype(jnp.float32).reshape(_NUM_IDX, 1)
  table128 = emb_table.reshape(_VOCAB // 2, 2 * _EMBED)
  gathered = _sc_gather(table128, pair_idx)
  hidden = _tc_reduce(gathered, parity)
  logits = _tc_project(hidden, W, b.reshape(1, _VOCAB))
  return logits


# speedup vs baseline: 2.3711x; 2.3711x over previous
import functools

import jax
import jax.numpy as jnp
from jax import lax
from jax.experimental import pallas as pl
from jax.experimental.pallas import tpu as pltpu
from jax.experimental.pallas import tpu_sc as plsc

_VOCAB = 100000
_EMBED = 64
_BATCH = 1024
_CTX = 20

_NUM_IDX = _BATCH * _CTX
_NC, _NS = 2, 16
_NW = _NC * _NS
_B_PER_W = _NUM_IDX // _NW

_P_BLK = 4096
_GRID_P = (_VOCAB + _P_BLK - 1) // _P_BLK

_V_BLK = 2048
_GRID_V = (_VOCAB + _V_BLK - 1) // _V_BLK


def _eye64():
  r = lax.broadcasted_iota(jnp.int32, (_EMBED, _EMBED), 0)
  c = lax.broadcasted_iota(jnp.int32, (_EMBED, _EMBED), 1)
  return (r == c).astype(jnp.float32)


def _pack_body(et_ref, o_ref):
  t = lax.dot_general(
      et_ref[...], _eye64(),
      dimension_numbers=(((0,), (0,)), ((), ())),
      precision=lax.Precision.HIGHEST,
      preferred_element_type=jnp.float32,
  )
  o_ref[...] = jnp.concatenate([t, t], axis=1)


def _pack(embT):
  return pl.pallas_call(
      _pack_body,
      grid=(_GRID_P,),
      in_specs=[pl.BlockSpec((_EMBED, _P_BLK), lambda i: (0, i))],
      out_specs=pl.BlockSpec((_P_BLK, 2 * _EMBED), lambda i: (i, 0)),
      out_shape=jax.ShapeDtypeStruct((_VOCAB, 2 * _EMBED), jnp.float32),
  )(embT)


def _sc_gather(packed, flat_idx):
  mesh = plsc.VectorSubcoreMesh(core_axis_name="c", subcore_axis_name="s")

  @functools.partial(
      pl.kernel,
      mesh=mesh,
      out_type=jax.ShapeDtypeStruct((_NUM_IDX, 2 * _EMBED), jnp.float32),
      scratch_types=[
          pltpu.VMEM((_B_PER_W,), jnp.int32),
          pltpu.VMEM((_B_PER_W, 2 * _EMBED), jnp.float32),
          pltpu.SemaphoreType.DMA,
      ],
  )
  def gather_kernel(table_hbm, idx_hbm, out_hbm, idx_v, rows_v, sem):
    wid = lax.axis_index("s") * _NC + lax.axis_index("c")
    base = wid * _B_PER_W
    pltpu.sync_copy(idx_hbm.at[pl.ds(base, _B_PER_W)], idx_v)
    pltpu.async_copy(table_hbm.at[idx_v], rows_v, sem).wait()
    pltpu.sync_copy(rows_v, out_hbm.at[pl.ds(base, _B_PER_W)])

  return gather_kernel(packed, flat_idx)


def _proj_body(g_ref, wt_ref, b_ref, o_ref, h_ref):
  i = pl.program_id(0)

  @pl.when(i == 0)
  def _():
    acc = g_ref[0:_BATCH, 0:_EMBED]
    for j in range(1, _CTX):
      acc = acc + g_ref[j * _BATCH:(j + 1) * _BATCH, 0:_EMBED]
    hT = lax.dot_general(
        _eye64(), acc,
        dimension_numbers=(((0,), (1,)), ((), ())),
        precision=lax.Precision.HIGHEST,
        preferred_element_type=jnp.float32,
    )
    h_ref[...] = hT * (1.0 / _CTX)

  o_ref[...] = lax.dot_general(
      wt_ref[...], h_ref[...],
      dimension_numbers=(((0,), (0,)), ((), ())),
      preferred_element_type=jnp.float32,
  ) + b_ref[...]


def _proj(gathered, WT, bT):
  return pl.pallas_call(
      _proj_body,
      grid=(_GRID_V,),
      in_specs=[
          pl.BlockSpec((_NUM_IDX, 2 * _EMBED), lambda i: (0, 0)),
          pl.BlockSpec((_EMBED, _V_BLK), lambda i: (0, i)),
          pl.BlockSpec((_V_BLK, 1), lambda i: (i, 0)),
      ],
      out_specs=pl.BlockSpec((_V_BLK, _BATCH), lambda i: (i, 0)),
      out_shape=jax.ShapeDtypeStruct((_VOCAB, _BATCH), jnp.float32),
      scratch_shapes=[pltpu.VMEM((_EMBED, _BATCH), jnp.float32)],
  )(gathered, WT, bT)


@jax.jit
def kernel(context, emb_table, W, b):
  flat_idx = context.T.reshape(-1)
  packed = _pack(emb_table.T)
  gathered = _sc_gather(packed, flat_idx)
  logitsT = _proj(gathered, W.T, b.reshape(_VOCAB, 1))
  return logitsT.T


# speedup vs baseline: 2.5963x; 1.0949x over previous
import functools

import jax
import jax.numpy as jnp
from jax import lax
from jax.experimental import pallas as pl
from jax.experimental.pallas import tpu as pltpu
from jax.experimental.pallas import tpu_sc as plsc

_VOCAB = 100000
_EMBED = 64
_BATCH = 1024
_CTX = 20

_NUM_IDX = _BATCH * _CTX
_NC, _NS = 2, 16
_NW = _NC * _NS
_B_PER_W = _NUM_IDX // _NW

_P_BLK = 4096
_GRID_P = (_VOCAB + _P_BLK - 1) // _P_BLK

_V_BLK = 2048
_GRID_V = (_VOCAB + _V_BLK - 1) // _V_BLK


def _eye64():
  r = lax.broadcasted_iota(jnp.int32, (_EMBED, _EMBED), 0)
  c = lax.broadcasted_iota(jnp.int32, (_EMBED, _EMBED), 1)
  return (r == c).astype(jnp.float32)


def _pack_body(et_ref, o_ref):
  t = lax.dot_general(
      et_ref[...], _eye64(),
      dimension_numbers=(((0,), (0,)), ((), ())),
      precision=lax.Precision.HIGHEST,
      preferred_element_type=jnp.float32,
  )
  o_ref[...] = jnp.concatenate([t, t], axis=1)


def _ones_row():
  return jnp.ones((1, _BATCH), dtype=jnp.float32)


def _pack(embT):
  return pl.pallas_call(
      _pack_body,
      grid=(_GRID_P,),
      in_specs=[pl.BlockSpec((_EMBED, _P_BLK), lambda i: (0, i))],
      out_specs=pl.BlockSpec((_P_BLK, 2 * _EMBED), lambda i: (i, 0)),
      out_shape=jax.ShapeDtypeStruct((_VOCAB, 2 * _EMBED), jnp.float32),
  )(embT)


def _sc_gather(packed, flat_idx):
  mesh = plsc.VectorSubcoreMesh(core_axis_name="c", subcore_axis_name="s")

  @functools.partial(
      pl.kernel,
      mesh=mesh,
      out_type=jax.ShapeDtypeStruct((_NUM_IDX, 2 * _EMBED), jnp.float32),
      scratch_types=[
          pltpu.VMEM((_B_PER_W,), jnp.int32),
          pltpu.VMEM((_B_PER_W, 2 * _EMBED), jnp.float32),
          pltpu.SemaphoreType.DMA,
      ],
  )
  def gather_kernel(table_hbm, idx_hbm, out_hbm, idx_v, rows_v, sem):
    wid = lax.axis_index("s") * _NC + lax.axis_index("c")
    base = wid * _B_PER_W
    pltpu.sync_copy(idx_hbm.at[pl.ds(base, _B_PER_W)], idx_v)
    pltpu.async_copy(table_hbm.at[idx_v], rows_v, sem).wait()
    pltpu.sync_copy(rows_v, out_hbm.at[pl.ds(base, _B_PER_W)])

  return gather_kernel(packed, flat_idx)


def _proj_body(g_ref, wt_ref, b_ref, o_ref, h_ref):
  i = pl.program_id(0)

  @pl.when(i == 0)
  def _():
    acc = g_ref[0:_BATCH, 0:_EMBED]
    for j in range(1, _CTX):
      acc = acc + g_ref[j * _BATCH:(j + 1) * _BATCH, 0:_EMBED]
    hT = lax.dot_general(
        _eye64(), acc,
        dimension_numbers=(((0,), (1,)), ((), ())),
        precision=lax.Precision.HIGHEST,
        preferred_element_type=jnp.float32,
    )
    h_ref[...] = hT * (1.0 / _CTX)

  bias = lax.dot_general(
      b_ref[...].reshape(1, _V_BLK), _ones_row(),
      dimension_numbers=(((0,), (0,)), ((), ())),
      precision=lax.Precision.HIGHEST,
      preferred_element_type=jnp.float32,
  )
  o_ref[...] = lax.dot_general(
      wt_ref[...], h_ref[...],
      dimension_numbers=(((0,), (0,)), ((), ())),
      preferred_element_type=jnp.float32,
  ) + bias


def _proj(gathered, WT, bT):
  return pl.pallas_call(
      _proj_body,
      grid=(_GRID_V,),
      in_specs=[
          pl.BlockSpec((_NUM_IDX, 2 * _EMBED), lambda i: (0, 0)),
          pl.BlockSpec((_EMBED, _V_BLK), lambda i: (0, i)),
          pl.BlockSpec((_V_BLK,), lambda i: (i,)),
      ],
      out_specs=pl.BlockSpec((_V_BLK, _BATCH), lambda i: (i, 0)),
      out_shape=jax.ShapeDtypeStruct((_VOCAB, _BATCH), jnp.float32),
      scratch_shapes=[pltpu.VMEM((_EMBED, _BATCH), jnp.float32)],
  )(gathered, WT, bT)


@jax.jit
def kernel(context, emb_table, W, b):
  flat_idx = context.T.reshape(-1)
  packed = _pack(emb_table.T)
  gathered = _sc_gather(packed, flat_idx)
  logitsT = _proj(gathered, W.T, b)
  return logitsT.T


# speedup vs baseline: 2.7154x; 1.0459x over previous
import functools

import jax
import jax.numpy as jnp
from jax import lax
from jax.experimental import pallas as pl
from jax.experimental.pallas import tpu as pltpu
from jax.experimental.pallas import tpu_sc as plsc

_VOCAB = 100000
_EMBED = 64
_BATCH = 1024
_CTX = 20

_NUM_IDX = _BATCH * _CTX
_NC, _NS = 2, 16
_NW = _NC * _NS
_B_PER_W = _NUM_IDX // _NW

_P_BLK = 4096
_GRID_P = (_VOCAB + _P_BLK - 1) // _P_BLK

_V_BLK = 2048
_GRID_V = (_VOCAB + _V_BLK - 1) // _V_BLK


def _eye64():
  r = lax.broadcasted_iota(jnp.int32, (_EMBED, _EMBED), 0)
  c = lax.broadcasted_iota(jnp.int32, (_EMBED, _EMBED), 1)
  return (r == c).astype(jnp.float32)


def _pack_body(et_ref, o_ref):
  t = jnp.swapaxes(et_ref[...], 0, 1)
  o_ref[...] = jnp.concatenate([t, t], axis=1)


def _ones_row():
  return jnp.ones((1, _BATCH), dtype=jnp.float32)


def _pack(embT):
  return pl.pallas_call(
      _pack_body,
      grid=(_GRID_P,),
      in_specs=[pl.BlockSpec((_EMBED, _P_BLK), lambda i: (0, i))],
      out_specs=pl.BlockSpec((_P_BLK, 2 * _EMBED), lambda i: (i, 0)),
      out_shape=jax.ShapeDtypeStruct((_VOCAB, 2 * _EMBED), jnp.float32),
  )(embT)


def _sc_gather(packed, flat_idx):
  mesh = plsc.VectorSubcoreMesh(core_axis_name="c", subcore_axis_name="s")

  @functools.partial(
      pl.kernel,
      mesh=mesh,
      out_type=jax.ShapeDtypeStruct((_NUM_IDX, 2 * _EMBED), jnp.float32),
      scratch_types=[
          pltpu.VMEM((_B_PER_W,), jnp.int32),
          pltpu.VMEM((_B_PER_W, 2 * _EMBED), jnp.float32),
          pltpu.SemaphoreType.DMA,
      ],
  )
  def gather_kernel(table_hbm, idx_hbm, out_hbm, idx_v, rows_v, sem):
    wid = lax.axis_index("s") * _NC + lax.axis_index("c")
    base = wid * _B_PER_W
    pltpu.sync_copy(idx_hbm.at[pl.ds(base, _B_PER_W)], idx_v)
    pltpu.async_copy(table_hbm.at[idx_v], rows_v, sem).wait()
    pltpu.sync_copy(rows_v, out_hbm.at[pl.ds(base, _B_PER_W)])

  return gather_kernel(packed, flat_idx)


def _proj_body(g_ref, wt_ref, b_ref, o_ref, h_ref):
  i = pl.program_id(0)

  @pl.when(i == 0)
  def _():
    acc = g_ref[0:_BATCH, 0:_EMBED]
    for j in range(1, _CTX):
      acc = acc + g_ref[j * _BATCH:(j + 1) * _BATCH, 0:_EMBED]
    hT = lax.dot_general(
        _eye64(), acc,
        dimension_numbers=(((0,), (1,)), ((), ())),
        precision=lax.Precision.HIGHEST,
        preferred_element_type=jnp.float32,
    )
    h_ref[...] = hT * (1.0 / _CTX)

  bias = lax.dot_general(
      b_ref[...].reshape(1, _V_BLK), _ones_row(),
      dimension_numbers=(((0,), (0,)), ((), ())),
      precision=lax.Precision.HIGHEST,
      preferred_element_type=jnp.float32,
  )
  o_ref[...] = lax.dot_general(
      wt_ref[...], h_ref[...],
      dimension_numbers=(((0,), (0,)), ((), ())),
      preferred_element_type=jnp.float32,
  ) + bias


def _proj(gathered, WT, bT):
  return pl.pallas_call(
      _proj_body,
      grid=(_GRID_V,),
      in_specs=[
          pl.BlockSpec((_NUM_IDX, 2 * _EMBED), lambda i: (0, 0)),
          pl.BlockSpec((_EMBED, _V_BLK), lambda i: (0, i)),
          pl.BlockSpec((_V_BLK,), lambda i: (i,)),
      ],
      out_specs=pl.BlockSpec((_V_BLK, _BATCH), lambda i: (i, 0)),
      out_shape=jax.ShapeDtypeStruct((_VOCAB, _BATCH), jnp.float32),
      scratch_shapes=[pltpu.VMEM((_EMBED, _BATCH), jnp.float32)],
  )(gathered, WT, bT)


@jax.jit
def kernel(context, emb_table, W, b):
  flat_idx = context.T.reshape(-1)
  packed = _pack(emb_table.T)
  gathered = _sc_gather(packed, flat_idx)
  logitsT = _proj(gathered, W.T, b)
  return logitsT.T


# speedup vs baseline: 3.1888x; 1.1743x over previous
import functools

import jax
import jax.numpy as jnp
from jax import lax
from jax.experimental import pallas as pl
from jax.experimental.pallas import tpu as pltpu
from jax.experimental.pallas import tpu_sc as plsc

_VOCAB = 100000
_EMBED = 64
_BATCH = 1024
_CTX = 20

_NUM_IDX = _BATCH * _CTX
_NC, _NS = 2, 16
_NW = _NC * _NS
_B_PER_W = _NUM_IDX // _NW

_P_BLK = 8192
_GRID_P = (_VOCAB + _P_BLK - 1) // _P_BLK

_V_BLK = 2048
_GRID_V = (_VOCAB + _V_BLK - 1) // _V_BLK


def _eye64():
  r = lax.broadcasted_iota(jnp.int32, (_EMBED, _EMBED), 0)
  c = lax.broadcasted_iota(jnp.int32, (_EMBED, _EMBED), 1)
  return (r == c).astype(jnp.float32)


def _pack_body(et_ref, o_ref):
  t = jnp.swapaxes(et_ref[...], 0, 1)
  o_ref[...] = jnp.concatenate([t, t], axis=1)


def _ones_row():
  return jnp.ones((1, _BATCH), dtype=jnp.float32)


def _pack(embT):
  return pl.pallas_call(
      _pack_body,
      grid=(_GRID_P,),
      in_specs=[pl.BlockSpec((_EMBED, _P_BLK), lambda i: (0, i))],
      out_specs=pl.BlockSpec((_P_BLK, 2 * _EMBED), lambda i: (i, 0)),
      out_shape=jax.ShapeDtypeStruct((_VOCAB, 2 * _EMBED), jnp.float32),
  )(embT)


def _sc_gather(packed, flat_idx):
  mesh = plsc.VectorSubcoreMesh(core_axis_name="c", subcore_axis_name="s")

  @functools.partial(
      pl.kernel,
      mesh=mesh,
      out_type=jax.ShapeDtypeStruct((_NUM_IDX, 2 * _EMBED), jnp.float32),
      scratch_types=[
          pltpu.VMEM((_B_PER_W,), jnp.int32),
          pltpu.VMEM((_B_PER_W, 2 * _EMBED), jnp.float32),
          pltpu.SemaphoreType.DMA,
      ],
  )
  def gather_kernel(table_hbm, idx_hbm, out_hbm, idx_v, rows_v, sem):
    wid = lax.axis_index("s") * _NC + lax.axis_index("c")
    base = wid * _B_PER_W
    pltpu.sync_copy(idx_hbm.at[pl.ds(base, _B_PER_W)], idx_v)
    pltpu.async_copy(table_hbm.at[idx_v], rows_v, sem).wait()
    pltpu.sync_copy(rows_v, out_hbm.at[pl.ds(base, _B_PER_W)])

  return gather_kernel(packed, flat_idx)


def _proj_body(g_ref, wt_ref, b_ref, o_ref, h_ref):
  i = pl.program_id(0)

  @pl.when(i == 0)
  def _():
    acc = g_ref[0:_BATCH, 0:_EMBED]
    for j in range(1, _CTX):
      acc = acc + g_ref[j * _BATCH:(j + 1) * _BATCH, 0:_EMBED]
    hT = lax.dot_general(
        _eye64(), acc,
        dimension_numbers=(((0,), (1,)), ((), ())),
        precision=lax.Precision.HIGHEST,
        preferred_element_type=jnp.float32,
    )
    h_ref[...] = hT * (1.0 / _CTX)

  bias = lax.dot_general(
      b_ref[...].reshape(1, _V_BLK), _ones_row(),
      dimension_numbers=(((0,), (0,)), ((), ())),
      preferred_element_type=jnp.float32,
  )
  o_ref[...] = lax.dot_general(
      wt_ref[...], h_ref[...],
      dimension_numbers=(((0,), (0,)), ((), ())),
      preferred_element_type=jnp.float32,
  ) + bias


def _proj(gathered, WT, bT):
  return pl.pallas_call(
      _proj_body,
      grid=(_GRID_V,),
      in_specs=[
          pl.BlockSpec((_NUM_IDX, 2 * _EMBED), lambda i: (0, 0)),
          pl.BlockSpec((_EMBED, _V_BLK), lambda i: (0, i)),
          pl.BlockSpec((_V_BLK,), lambda i: (i,)),
      ],
      out_specs=pl.BlockSpec((_V_BLK, _BATCH), lambda i: (i, 0)),
      out_shape=jax.ShapeDtypeStruct((_VOCAB, _BATCH), jnp.float32),
      scratch_shapes=[pltpu.VMEM((_EMBED, _BATCH), jnp.float32)],
  )(gathered, WT, bT)


@jax.jit
def kernel(context, emb_table, W, b):
  flat_idx = context.T.reshape(-1)
  packed = _pack(emb_table.T)
  gathered = _sc_gather(packed, flat_idx)
  logitsT = _proj(gathered, W.T, b)
  return logitsT.T


# speedup vs baseline: 3.2581x; 1.0217x over previous
import dataclasses
import functools

import jax
import jax.numpy as jnp
from jax import lax
from jax.experimental import pallas as pl
from jax.experimental.pallas import tpu as pltpu
from jax.experimental.pallas import tpu_sc as plsc

_VOCAB = 100000
_EMBED = 64
_BATCH = 1024
_CTX = 20

_NUM_IDX = _BATCH * _CTX
_NC, _NS = 2, 16
_NW = _NC * _NS
_V_BLK = 2048
_GRID_V = (_VOCAB + _V_BLK - 1) // _V_BLK


def _ones_row():
  return jnp.ones((1, _BATCH), dtype=jnp.float32)


def _sc_hidden(embT, flat_idx):
  mesh = plsc.VectorSubcoreMesh(core_axis_name="c", subcore_axis_name="s")
  _CHUNKS = _BATCH // 16

  cp = pltpu.CompilerParams()
  if "needs_layout_passes" in pltpu.CompilerParams.__dataclass_fields__:
    cp = dataclasses.replace(cp, needs_layout_passes=False)

  @functools.partial(
      pl.kernel,
      mesh=mesh,
      compiler_params=cp,
      out_type=jax.ShapeDtypeStruct((_EMBED, _BATCH), jnp.float32),
      scratch_types=[
          pltpu.VMEM((_NUM_IDX,), jnp.int32),
          pltpu.VMEM((_VOCAB,), jnp.float32),
          pltpu.VMEM((_BATCH,), jnp.float32),
          pltpu.SemaphoreType.DMA,
      ],
  )
  def hidden_kernel(embT_hbm, idx_hbm, out_hbm, idx_v, row_v, acc_v, sem):
    wid = lax.axis_index("s") * _NC + lax.axis_index("c")
    pltpu.sync_copy(idx_hbm, idx_v)

    @pl.loop(0, 2)
    def _(t):
      e = wid + t * _NW
      pltpu.sync_copy(embT_hbm.at[e], row_v)

      @pl.loop(0, _CHUNKS // 4)
      def _(cc):
        for k in range(4):
          s = (cc * 4 + k) * 16
          iv = idx_v[pl.ds(s, 16)]
          acc_v[pl.ds(s, 16)] = plsc.load_gather(row_v, [iv])

      @pl.loop(1, _CTX)
      def _(j):
        @pl.loop(0, _CHUNKS // 4)
        def _(cc):
          for k in range(4):
            s = (cc * 4 + k) * 16
            iv = idx_v[pl.ds(j * _BATCH + s, 16)]
            g = plsc.load_gather(row_v, [iv])
            acc_v[pl.ds(s, 16)] = acc_v[pl.ds(s, 16)] + g

      @pl.loop(0, _CHUNKS // 4)
      def _(cc):
        for k in range(4):
          s = (cc * 4 + k) * 16
          acc_v[pl.ds(s, 16)] = acc_v[pl.ds(s, 16)] * (1.0 / _CTX)

      pltpu.sync_copy(acc_v, out_hbm.at[e])

  return hidden_kernel(embT, flat_idx)


def _proj_body(h_ref, wt_ref, b_ref, o_ref):
  bias = lax.dot_general(
      b_ref[...].reshape(1, _V_BLK), _ones_row(),
      dimension_numbers=(((0,), (0,)), ((), ())),
      preferred_element_type=jnp.float32,
  )
  o_ref[...] = lax.dot_general(
      wt_ref[...], h_ref[...],
      dimension_numbers=(((0,), (0,)), ((), ())),
      preferred_element_type=jnp.float32,
  ) + bias


def _proj(hT, WT, b):
  return pl.pallas_call(
      _proj_body,
      grid=(_GRID_V,),
      in_specs=[
          pl.BlockSpec((_EMBED, _BATCH), lambda i: (0, 0)),
          pl.BlockSpec((_EMBED, _V_BLK), lambda i: (0, i)),
          pl.BlockSpec((_V_BLK,), lambda i: (i,)),
      ],
      out_specs=pl.BlockSpec((_V_BLK, _BATCH), lambda i: (i, 0)),
      out_shape=jax.ShapeDtypeStruct((_VOCAB, _BATCH), jnp.float32),
  )(hT, WT, b)


@jax.jit
def kernel(context, emb_table, W, b):
  flat_idx = context.T.reshape(-1)
  hT = _sc_hidden(emb_table.T, flat_idx)
  logitsT = _proj(hT, W.T, b)
  return logitsT.T


# speedup vs baseline: 3.2959x; 1.0116x over previous
import dataclasses
import functools

import jax
import jax.numpy as jnp
from jax import lax
from jax.experimental import pallas as pl
from jax.experimental.pallas import tpu as pltpu
from jax.experimental.pallas import tpu_sc as plsc

_VOCAB = 100000
_EMBED = 64
_BATCH = 1024
_CTX = 20

_NUM_IDX = _BATCH * _CTX
_NC, _NS = 2, 16
_NW = _NC * _NS
_V_BLK = 4096
_GRID_V = (_VOCAB + _V_BLK - 1) // _V_BLK


def _ones_row():
  return jnp.ones((1, _BATCH), dtype=jnp.float32)


def _sc_hidden(embT, flat_idx):
  mesh = plsc.VectorSubcoreMesh(core_axis_name="c", subcore_axis_name="s")
  _CHUNKS = _BATCH // 16

  cp = pltpu.CompilerParams()
  if "needs_layout_passes" in pltpu.CompilerParams.__dataclass_fields__:
    cp = dataclasses.replace(cp, needs_layout_passes=False)

  @functools.partial(
      pl.kernel,
      mesh=mesh,
      compiler_params=cp,
      out_type=jax.ShapeDtypeStruct((_EMBED, _BATCH), jnp.float32),
      scratch_types=[
          pltpu.VMEM((_NUM_IDX,), jnp.int32),
          pltpu.VMEM((_VOCAB,), jnp.float32),
          pltpu.VMEM((_BATCH,), jnp.float32),
          pltpu.SemaphoreType.DMA,
      ],
  )
  def hidden_kernel(embT_hbm, idx_hbm, out_hbm, idx_v, row_v, acc_v, sem):
    wid = lax.axis_index("s") * _NC + lax.axis_index("c")
    pltpu.sync_copy(idx_hbm, idx_v)

    @pl.loop(0, 2)
    def _(t):
      e = wid + t * _NW
      pltpu.sync_copy(embT_hbm.at[e], row_v)

      @pl.loop(0, _CHUNKS // 4)
      def _(cc):
        for k in range(4):
          s = (cc * 4 + k) * 16
          iv = idx_v[pl.ds(s, 16)]
          acc_v[pl.ds(s, 16)] = plsc.load_gather(row_v, [iv])

      @pl.loop(1, _CTX)
      def _(j):
        @pl.loop(0, _CHUNKS // 4)
        def _(cc):
          for k in range(4):
            s = (cc * 4 + k) * 16
            iv = idx_v[pl.ds(j * _BATCH + s, 16)]
            g = plsc.load_gather(row_v, [iv])
            acc_v[pl.ds(s, 16)] = acc_v[pl.ds(s, 16)] + g

      @pl.loop(0, _CHUNKS // 4)
      def _(cc):
        for k in range(4):
          s = (cc * 4 + k) * 16
          acc_v[pl.ds(s, 16)] = acc_v[pl.ds(s, 16)] * (1.0 / _CTX)

      pltpu.sync_copy(acc_v, out_hbm.at[e])

  return hidden_kernel(embT, flat_idx)


def _proj_body(h_ref, wt_ref, b_ref, o_ref):
  bias = lax.dot_general(
      b_ref[...].reshape(1, _V_BLK), _ones_row(),
      dimension_numbers=(((0,), (0,)), ((), ())),
      preferred_element_type=jnp.float32,
  )
  o_ref[...] = lax.dot_general(
      wt_ref[...], h_ref[...],
      dimension_numbers=(((0,), (0,)), ((), ())),
      preferred_element_type=jnp.float32,
  ) + bias


def _proj(hT, WT, b):
  return pl.pallas_call(
      _proj_body,
      grid=(_GRID_V,),
      in_specs=[
          pl.BlockSpec((_EMBED, _BATCH), lambda i: (0, 0)),
          pl.BlockSpec((_EMBED, _V_BLK), lambda i: (0, i)),
          pl.BlockSpec((_V_BLK,), lambda i: (i,)),
      ],
      out_specs=pl.BlockSpec((_V_BLK, _BATCH), lambda i: (i, 0)),
      out_shape=jax.ShapeDtypeStruct((_VOCAB, _BATCH), jnp.float32),
  )(hT, WT, b)


@jax.jit
def kernel(context, emb_table, W, b):
  flat_idx = context.T.reshape(-1)
  hT = _sc_hidden(emb_table.T, flat_idx)
  logitsT = _proj(hT, W.T, b)
  return logitsT.T
